# baseline (speedup 1.0000x reference)
import functools

import jax
import jax.numpy as jnp
from jax import lax
from jax.experimental import pallas as pl
from jax.experimental.pallas import tpu as pltpu
from jax.experimental.pallas import tpu_sc as plsc

N = 10000
E = 320000
D = 128
DH = D // 2
DOUT = 256

NPAD = 10240
NW = 32
CH = 256
K = 40
EPAD = NW * K * CH
ROWS_PER_TILE = NPAD // 16
NBUF = 4
ZROWS = 32

_mesh_kw = dict(core_axis_name="c", subcore_axis_name="s",
                num_cores=2, num_subcores=16)


def _count_body(dst_hbm, cnt_hbm, idx_v, buf_v, cnt_sh, sem):
    c = lax.axis_index("c")
    s = lax.axis_index("s")
    w = c * 16 + s

    def _zero_row(i, _):
        buf_v[i, :] = jnp.zeros((16,), jnp.float32)
        return 0

    lax.fori_loop(0, ROWS_PER_TILE, _zero_row, 0)
    pltpu.sync_copy(buf_v, cnt_sh.at[pl.ds(s * ROWS_PER_TILE, ROWS_PER_TILE)])

    pltpu.sync_copy(dst_hbm.at[w], idx_v)

    def _one_row(i, _):
        buf_v[i, :] = jnp.ones((16,), jnp.float32)
        return 0

    lax.fori_loop(0, CH, _one_row, 0)

    plsc.subcore_barrier()

    def _body(j, _):
        pltpu.async_copy(buf_v.at[pl.ds(0, CH)], cnt_sh.at[idx_v.at[j]],
                         sem, add=True)
        return 0

    lax.fori_loop(0, K, _body, 0)

    def _drain(j, _):
        pltpu.make_async_copy(buf_v.at[pl.ds(0, CH)],
                              cnt_sh.at[idx_v.at[j]], sem).wait()
        return 0

    lax.fori_loop(0, K, _drain, 0)

    plsc.subcore_barrier()

    pltpu.sync_copy(
        cnt_sh.at[pl.ds(s * ROWS_PER_TILE, ROWS_PER_TILE)],
        cnt_hbm.at[c, pl.ds(s * ROWS_PER_TILE, ROWS_PER_TILE)],
    )


def _agg_body(src_hbm, dst_hbm, hp_lo_hbm, hp_hi_hbm, acc_hbm,
              sidx, didx, rows, zbuf, acc_sh, gsem, ssem):
    c = lax.axis_index("c")
    s = lax.axis_index("s")
    w = c * 16 + s

    def _zero_row(i, _):
        def _zero_lane(jj, _):
            zbuf[i, pl.ds(jj * 16, 16)] = jnp.zeros((16,), jnp.float32)
            return 0
        lax.fori_loop(0, DH // 16, _zero_lane, 0)
        return 0

    lax.fori_loop(0, ZROWS, _zero_row, 0)

    pltpu.sync_copy(src_hbm.at[w], sidx)
    pltpu.sync_copy(dst_hbm.at[w], didx)

    for p, hp_hbm in ((0, hp_lo_hbm), (1, hp_hi_hbm)):
        def _zero_acc(k, _):
            pltpu.async_copy(
                zbuf, acc_sh.at[pl.ds(s * ROWS_PER_TILE + k * ZROWS, ZROWS)],
                gsem)
            return 0

        lax.fori_loop(0, ROWS_PER_TILE // ZROWS, _zero_acc, 0)

        def _zero_drain(k, _):
            pltpu.make_async_copy(
                zbuf, acc_sh.at[pl.ds(s * ROWS_PER_TILE + k * ZROWS, ZROWS)],
                gsem).wait()
            return 0

        lax.fori_loop(0, ROWS_PER_TILE // ZROWS, _zero_drain, 0)
        plsc.subcore_barrier()

        for q in range(NBUF - 1):
            pltpu.async_copy(hp_hbm.at[sidx.at[q]], rows.at[q], gsem)

        def _body(j, _):
            b = lax.rem(j, NBUF)
            pltpu.make_async_copy(hp_hbm.at[sidx.at[j]], rows.at[b],
                                  gsem).wait()
            pltpu.async_copy(rows.at[b], acc_sh.at[didx.at[j]], ssem,
                             add=True)

            @pl.when(j >= 1)
            def _():
                pltpu.make_async_copy(rows.at[lax.rem(j - 1, NBUF)],
                                      acc_sh.at[didx.at[j - 1]], ssem).wait()

            @pl.when(j < K - (NBUF - 1))
            def _():
                pltpu.async_copy(hp_hbm.at[sidx.at[j + NBUF - 1]],
                                 rows.at[lax.rem(j + NBUF - 1, NBUF)], gsem)

            return 0

        lax.fori_loop(0, K, _body, 0)
        pltpu.make_async_copy(rows.at[(K - 1) % NBUF],
                              acc_sh.at[didx.at[K - 1]], ssem).wait()
        plsc.subcore_barrier()

        pltpu.sync_copy(
            acc_sh.at[pl.ds(s * ROWS_PER_TILE, ROWS_PER_TILE)],
            acc_hbm.at[c, p, pl.ds(s * ROWS_PER_TILE, ROWS_PER_TILE)],
        )


@functools.cache
def _sc_kernels():
    mesh = plsc.VectorSubcoreMesh(**_mesh_kw)
    count_kernel = pl.kernel(
        _count_body,
        out_type=jax.ShapeDtypeStruct((2, NPAD, 16), jnp.float32),
        mesh=mesh,
        compiler_params=pltpu.CompilerParams(use_tc_tiling_on_sc=False),
        scratch_types=[
            pltpu.VMEM((K, CH), jnp.int32),
            pltpu.VMEM((ROWS_PER_TILE, 16), jnp.float32),
            pltpu.VMEM_SHARED((NPAD, 16), jnp.float32),
            pltpu.SemaphoreType.DMA,
        ],
    )
    agg_kernel = pl.kernel(
        _agg_body,
        out_type=jax.ShapeDtypeStruct((2, 2, NPAD, DH), jnp.float32),
        mesh=mesh,
        compiler_params=pltpu.CompilerParams(use_tc_tiling_on_sc=False),
        scratch_types=[
            pltpu.VMEM((K, CH), jnp.int32),
            pltpu.VMEM((K, CH), jnp.int32),
            pltpu.VMEM((NBUF, CH, DH), jnp.float32),
            pltpu.VMEM((ZROWS, DH), jnp.float32),
            pltpu.VMEM_SHARED((NPAD, DH), jnp.float32),
            pltpu.SemaphoreType.DMA,
            pltpu.SemaphoreType.DMA,
        ],
    )
    return count_kernel, agg_kernel


def _dense1_body(x_ref, w1_ref, cnt_ref, hp_lo_ref, hp_hi_ref, dis_ref):
    deg = cnt_ref[0][:, 0:1] + cnt_ref[1][:, 0:1] + 1.0
    dis = lax.rsqrt(deg)
    h = jnp.dot(x_ref[...], w1_ref[...], preferred_element_type=jnp.float32)
    hp = h * dis
    hp_lo_ref[...] = hp[:, :DH]
    hp_hi_ref[...] = hp[:, DH:]
    dis_ref[...] = dis


def _dense1(x_pad, W1, cnt):
    bs = 1024
    grid = NPAD // bs
    return pl.pallas_call(
        _dense1_body,
        grid=(grid,),
        in_specs=[
            pl.BlockSpec((bs, D), lambda i: (i, 0)),
            pl.BlockSpec((D, D), lambda i: (0, 0)),
            pl.BlockSpec((2, bs, 16), lambda i: (0, i, 0)),
        ],
        out_specs=[
            pl.BlockSpec((bs, DH), lambda i: (i, 0)),
            pl.BlockSpec((bs, DH), lambda i: (i, 0)),
            pl.BlockSpec((bs, 1), lambda i: (i, 0)),
        ],
        out_shape=[
            jax.ShapeDtypeStruct((NPAD, DH), jnp.float32),
            jax.ShapeDtypeStruct((NPAD, DH), jnp.float32),
            jax.ShapeDtypeStruct((NPAD, 1), jnp.float32),
        ],
    )(x_pad, W1, cnt)


def _dense2_body(acc_ref, hp_lo_ref, hp_hi_ref, dis_ref, b1_ref, gamma_ref,
                 beta_ref, w2_ref, b2_ref, out_ref):
    hp = jnp.concatenate([hp_lo_ref[...], hp_hi_ref[...]], axis=-1)
    agg = jnp.concatenate(
        [acc_ref[0, 0] + acc_ref[1, 0], acc_ref[0, 1] + acc_ref[1, 1]],
        axis=-1)
    g = (agg + hp) * dis_ref[...]
    z = jnp.maximum(g + b1_ref[...], 0.0)
    mu = jnp.mean(z, axis=-1, keepdims=True)
    zc = z - mu
    var = jnp.mean(zc * zc, axis=-1, keepdims=True)
    zn = zc * lax.rsqrt(var + 1e-5) * gamma_ref[...] + beta_ref[...]
    y = jnp.dot(zn, w2_ref[...], preferred_element_type=jnp.float32)
    out_ref[...] = jnp.maximum(y + b2_ref[...], 0.0)


def _dense2(acc, hp_lo, hp_hi, dis, b1, gamma, beta, W2, b2):
    bs = 512
    grid = NPAD // bs
    return pl.pallas_call(
        _dense2_body,
        grid=(grid,),
        in_specs=[
            pl.BlockSpec((2, 2, bs, DH), lambda i: (0, 0, i, 0)),
            pl.BlockSpec((bs, DH), lambda i: (i, 0)),
            pl.BlockSpec((bs, DH), lambda i: (i, 0)),
            pl.BlockSpec((bs, 1), lambda i: (i, 0)),
            pl.BlockSpec((D,), lambda i: (0,)),
            pl.BlockSpec((D,), lambda i: (0,)),
            pl.BlockSpec((D,), lambda i: (0,)),
            pl.BlockSpec((D, DOUT), lambda i: (0, 0)),
            pl.BlockSpec((DOUT,), lambda i: (0,)),
        ],
        out_specs=pl.BlockSpec((bs, DOUT), lambda i: (i, 0)),
        out_shape=jax.ShapeDtypeStruct((NPAD, DOUT), jnp.float32),
    )(acc, hp_lo, hp_hi, dis, b1, gamma, beta, W2, b2)


def kernel(x, edge_index, W1, b1, gamma, beta, W2, b2):
    ei = edge_index.astype(jnp.int32)
    src = ei[0]
    dst = ei[1]
    pad = EPAD - E
    src3 = jnp.concatenate(
        [src, jnp.zeros((pad,), jnp.int32)]).reshape(NW, K, CH)
    trash = N + jax.lax.rem(jnp.arange(pad, dtype=jnp.int32),
                            jnp.int32(NPAD - N))
    dst3 = jnp.concatenate([dst, trash]).reshape(NW, K, CH)
    x_pad = jnp.pad(x, ((0, NPAD - N), (0, 0)))

    count_kernel, agg_kernel = _sc_kernels()
    cnt = count_kernel(dst3)
    hp_lo, hp_hi, dis = _dense1(x_pad, W1, cnt)
    acc = agg_kernel(src3, dst3, hp_lo, hp_hi)
    out = _dense2(acc, hp_lo, hp_hi, dis, b1, gamma, beta, W2, b2)
    return (out[:N], edge_index)


# speedup vs baseline: 1.4846x; 1.4846x over previous
import functools

import jax
import jax.numpy as jnp
from jax import lax
from jax.experimental import pallas as pl
from jax.experimental.pallas import tpu as pltpu
from jax.experimental.pallas import tpu_sc as plsc

N = 10000
E = 320000
D = 128
DH = D // 2
DOUT = 256

NPAD = 10240
NW = 32
CH = 128
K = 79
EPAD = NW * K * CH
ROWS_PER_TILE = NPAD // 16
NBUF = 4
ZROWS = 32

_mesh_kw = dict(core_axis_name="c", subcore_axis_name="s",
                num_cores=2, num_subcores=16)


def _count_body(dst_hbm, cnt_hbm, idx_v, buf_v, cnt_sh, sem):
    c = lax.axis_index("c")
    s = lax.axis_index("s")
    w = c * 16 + s

    def _zero_row(i, _):
        buf_v[i, :] = jnp.zeros((16,), jnp.float32)
        return 0

    lax.fori_loop(0, ROWS_PER_TILE, _zero_row, 0)
    pltpu.sync_copy(buf_v, cnt_sh.at[pl.ds(s * ROWS_PER_TILE, ROWS_PER_TILE)])

    pltpu.sync_copy(dst_hbm.at[w], idx_v)

    def _one_row(i, _):
        buf_v[i, :] = jnp.ones((16,), jnp.float32)
        return 0

    lax.fori_loop(0, CH, _one_row, 0)

    plsc.subcore_barrier()

    def _body(j, _):
        pltpu.async_copy(buf_v.at[pl.ds(0, CH)], cnt_sh.at[idx_v.at[j]],
                         sem, add=True)
        return 0

    lax.fori_loop(0, K, _body, 0)

    def _drain(j, _):
        pltpu.make_async_copy(buf_v.at[pl.ds(0, CH)],
                              cnt_sh.at[idx_v.at[j]], sem).wait()
        return 0

    lax.fori_loop(0, K, _drain, 0)

    plsc.subcore_barrier()

    pltpu.sync_copy(
        cnt_sh.at[pl.ds(s * ROWS_PER_TILE, ROWS_PER_TILE)],
        cnt_hbm.at[c, pl.ds(s * ROWS_PER_TILE, ROWS_PER_TILE)],
    )


def _agg_body(src_hbm, dst_hbm, hp_lo_hbm, hp_hi_hbm, acc_hbm,
              sidx, didx, rows, zbuf, acc_sh, gsem, ssem):
    c = lax.axis_index("c")
    s = lax.axis_index("s")
    w = c * 16 + s

    def _zero_row(i, _):
        def _zero_lane(jj, _):
            zbuf[i, pl.ds(jj * 16, 16)] = jnp.zeros((16,), jnp.float32)
            return 0
        lax.fori_loop(0, DH // 16, _zero_lane, 0)
        return 0

    lax.fori_loop(0, ZROWS, _zero_row, 0)

    pltpu.sync_copy(src_hbm.at[w], sidx)
    pltpu.sync_copy(dst_hbm.at[w], didx)

    for p, hp_hbm in ((0, hp_lo_hbm), (1, hp_hi_hbm)):
        def _zero_acc(k, _):
            pltpu.async_copy(
                zbuf, acc_sh.at[pl.ds(s * ROWS_PER_TILE + k * ZROWS, ZROWS)],
                gsem)
            return 0

        lax.fori_loop(0, ROWS_PER_TILE // ZROWS, _zero_acc, 0)

        def _zero_drain(k, _):
            pltpu.make_async_copy(
                zbuf, acc_sh.at[pl.ds(s * ROWS_PER_TILE + k * ZROWS, ZROWS)],
                gsem).wait()
            return 0

        lax.fori_loop(0, ROWS_PER_TILE // ZROWS, _zero_drain, 0)
        plsc.subcore_barrier()

        for q in range(NBUF - 1):
            pltpu.async_copy(hp_hbm.at[sidx.at[q]], rows.at[q], gsem)

        def _body(j, _):
            b = lax.rem(j, NBUF)
            pltpu.make_async_copy(hp_hbm.at[sidx.at[j]], rows.at[b],
                                  gsem).wait()
            pltpu.async_copy(rows.at[b], acc_sh.at[didx.at[j]], ssem,
                             add=True)

            @pl.when(j >= 1)
            def _():
                pltpu.make_async_copy(rows.at[lax.rem(j - 1, NBUF)],
                                      acc_sh.at[didx.at[j - 1]], ssem).wait()

            @pl.when(j < K - (NBUF - 1))
            def _():
                pltpu.async_copy(hp_hbm.at[sidx.at[j + NBUF - 1]],
                                 rows.at[lax.rem(j + NBUF - 1, NBUF)], gsem)

            return 0

        lax.fori_loop(0, K, _body, 0)
        pltpu.make_async_copy(rows.at[(K - 1) % NBUF],
                              acc_sh.at[didx.at[K - 1]], ssem).wait()
        plsc.subcore_barrier()

        pltpu.sync_copy(
            acc_sh.at[pl.ds(s * ROWS_PER_TILE, ROWS_PER_TILE)],
            acc_hbm.at[c, p, pl.ds(s * ROWS_PER_TILE, ROWS_PER_TILE)],
        )


@functools.cache
def _sc_kernels():
    mesh = plsc.VectorSubcoreMesh(**_mesh_kw)
    count_kernel = pl.kernel(
        _count_body,
        out_type=jax.ShapeDtypeStruct((2, NPAD, 16), jnp.float32),
        mesh=mesh,
        compiler_params=pltpu.CompilerParams(use_tc_tiling_on_sc=False),
        scratch_types=[
            pltpu.VMEM((K, CH), jnp.int32),
            pltpu.VMEM((ROWS_PER_TILE, 16), jnp.float32),
            pltpu.VMEM_SHARED((NPAD, 16), jnp.float32),
            pltpu.SemaphoreType.DMA,
        ],
    )
    agg_kernel = pl.kernel(
        _agg_body,
        out_type=jax.ShapeDtypeStruct((2, 2, NPAD, DH), jnp.float32),
        mesh=mesh,
        compiler_params=pltpu.CompilerParams(use_tc_tiling_on_sc=False),
        scratch_types=[
            pltpu.VMEM((K, CH), jnp.int32),
            pltpu.VMEM((K, CH), jnp.int32),
            pltpu.VMEM((NBUF, CH, DH), jnp.float32),
            pltpu.VMEM((ZROWS, DH), jnp.float32),
            pltpu.VMEM_SHARED((NPAD, DH), jnp.float32),
            pltpu.SemaphoreType.DMA,
            pltpu.SemaphoreType.DMA,
        ],
    )
    return count_kernel, agg_kernel


def _dense1_body(x_ref, w1_ref, cnt_ref, hp_lo_ref, hp_hi_ref, dis_ref):
    deg = cnt_ref[0][:, 0:1] + cnt_ref[1][:, 0:1] + 1.0
    dis = lax.rsqrt(deg)
    h = jnp.dot(x_ref[...], w1_ref[...], preferred_element_type=jnp.float32)
    hp = h * dis
    hp_lo_ref[...] = hp[:, :DH]
    hp_hi_ref[...] = hp[:, DH:]
    dis_ref[...] = dis


def _dense1(x_pad, W1, cnt):
    bs = 1024
    grid = NPAD // bs
    return pl.pallas_call(
        _dense1_body,
        grid=(grid,),
        in_specs=[
            pl.BlockSpec((bs, D), lambda i: (i, 0)),
            pl.BlockSpec((D, D), lambda i: (0, 0)),
            pl.BlockSpec((2, bs, 16), lambda i: (0, i, 0)),
        ],
        out_specs=[
            pl.BlockSpec((bs, DH), lambda i: (i, 0)),
            pl.BlockSpec((bs, DH), lambda i: (i, 0)),
            pl.BlockSpec((bs, 1), lambda i: (i, 0)),
        ],
        out_shape=[
            jax.ShapeDtypeStruct((NPAD, DH), jnp.float32),
            jax.ShapeDtypeStruct((NPAD, DH), jnp.float32),
            jax.ShapeDtypeStruct((NPAD, 1), jnp.float32),
        ],
    )(x_pad, W1, cnt)


def _dense2_body(acc_ref, hp_lo_ref, hp_hi_ref, dis_ref, b1_ref, gamma_ref,
                 beta_ref, w2_ref, b2_ref, out_ref):
    hp = jnp.concatenate([hp_lo_ref[...], hp_hi_ref[...]], axis=-1)
    agg = jnp.concatenate(
        [acc_ref[0, 0] + acc_ref[1, 0], acc_ref[0, 1] + acc_ref[1, 1]],
        axis=-1)
    g = (agg + hp) * dis_ref[...]
    z = jnp.maximum(g + b1_ref[...], 0.0)
    mu = jnp.mean(z, axis=-1, keepdims=True)
    zc = z - mu
    var = jnp.mean(zc * zc, axis=-1, keepdims=True)
    zn = zc * lax.rsqrt(var + 1e-5) * gamma_ref[...] + beta_ref[...]
    y = jnp.dot(zn, w2_ref[...], preferred_element_type=jnp.float32)
    out_ref[...] = jnp.maximum(y + b2_ref[...], 0.0)


def _dense2(acc, hp_lo, hp_hi, dis, b1, gamma, beta, W2, b2):
    bs = 512
    grid = NPAD // bs
    return pl.pallas_call(
        _dense2_body,
        grid=(grid,),
        in_specs=[
            pl.BlockSpec((2, 2, bs, DH), lambda i: (0, 0, i, 0)),
            pl.BlockSpec((bs, DH), lambda i: (i, 0)),
            pl.BlockSpec((bs, DH), lambda i: (i, 0)),
            pl.BlockSpec((bs, 1), lambda i: (i, 0)),
            pl.BlockSpec((D,), lambda i: (0,)),
            pl.BlockSpec((D,), lambda i: (0,)),
            pl.BlockSpec((D,), lambda i: (0,)),
            pl.BlockSpec((D, DOUT), lambda i: (0, 0)),
            pl.BlockSpec((DOUT,), lambda i: (0,)),
        ],
        out_specs=pl.BlockSpec((bs, DOUT), lambda i: (i, 0)),
        out_shape=jax.ShapeDtypeStruct((NPAD, DOUT), jnp.float32),
    )(acc, hp_lo, hp_hi, dis, b1, gamma, beta, W2, b2)


def kernel(x, edge_index, W1, b1, gamma, beta, W2, b2):
    ei = edge_index.astype(jnp.int32)
    src = ei[0]
    dst = ei[1]
    pad = EPAD - E
    src3 = jnp.concatenate(
        [src, jnp.zeros((pad,), jnp.int32)]).reshape(NW, K, CH)
    trash = N + jax.lax.rem(jnp.arange(pad, dtype=jnp.int32),
                            jnp.int32(NPAD - N))
    dst3 = jnp.concatenate([dst, trash]).reshape(NW, K, CH)
    x_pad = jnp.pad(x, ((0, NPAD - N), (0, 0)))

    count_kernel, agg_kernel = _sc_kernels()
    cnt = count_kernel(dst3)
    hp_lo, hp_hi, dis = _dense1(x_pad, W1, cnt)
    acc = agg_kernel(src3, dst3, hp_lo, hp_hi)
    out = _dense2(acc, hp_lo, hp_hi, dis, b1, gamma, beta, W2, b2)
    return (out[:N], edge_index)


# speedup vs baseline: 1.4898x; 1.0036x over previous
import functools

import jax
import jax.numpy as jnp
from jax import lax
from jax.experimental import pallas as pl
from jax.experimental.pallas import tpu as pltpu
from jax.experimental.pallas import tpu_sc as plsc

N = 10000
E = 320000
D = 128
DH = D // 2
DOUT = 256

NPAD = 10240
NW = 32
CH = 128
K = 79
EPAD = NW * K * CH
ROWS_PER_TILE = NPAD // 16
NBUF = 8
ZROWS = 32

_mesh_kw = dict(core_axis_name="c", subcore_axis_name="s",
                num_cores=2, num_subcores=16)


def _count_body(dst_hbm, cnt_hbm, idx_v, buf_v, cnt_sh, sem):
    c = lax.axis_index("c")
    s = lax.axis_index("s")
    w = c * 16 + s

    def _zero_row(i, _):
        buf_v[i, :] = jnp.zeros((16,), jnp.float32)
        return 0

    lax.fori_loop(0, ROWS_PER_TILE, _zero_row, 0)
    pltpu.sync_copy(buf_v, cnt_sh.at[pl.ds(s * ROWS_PER_TILE, ROWS_PER_TILE)])

    pltpu.sync_copy(dst_hbm.at[w], idx_v)

    def _one_row(i, _):
        buf_v[i, :] = jnp.ones((16,), jnp.float32)
        return 0

    lax.fori_loop(0, CH, _one_row, 0)

    plsc.subcore_barrier()

    def _body(j, _):
        pltpu.async_copy(buf_v.at[pl.ds(0, CH)], cnt_sh.at[idx_v.at[j]],
                         sem, add=True)
        return 0

    lax.fori_loop(0, K, _body, 0)

    def _drain(j, _):
        pltpu.make_async_copy(buf_v.at[pl.ds(0, CH)],
                              cnt_sh.at[idx_v.at[j]], sem).wait()
        return 0

    lax.fori_loop(0, K, _drain, 0)

    plsc.subcore_barrier()

    pltpu.sync_copy(
        cnt_sh.at[pl.ds(s * ROWS_PER_TILE, ROWS_PER_TILE)],
        cnt_hbm.at[c, pl.ds(s * ROWS_PER_TILE, ROWS_PER_TILE)],
    )


def _agg_body(src_hbm, dst_hbm, hp_lo_hbm, hp_hi_hbm, acc_hbm,
              sidx, didx, rows, zbuf, acc_sh, gsem, ssem):
    c = lax.axis_index("c")
    s = lax.axis_index("s")
    w = c * 16 + s

    def _zero_row(i, _):
        def _zero_lane(jj, _):
            zbuf[i, pl.ds(jj * 16, 16)] = jnp.zeros((16,), jnp.float32)
            return 0
        lax.fori_loop(0, DH // 16, _zero_lane, 0)
        return 0

    lax.fori_loop(0, ZROWS, _zero_row, 0)

    pltpu.sync_copy(src_hbm.at[w], sidx)
    pltpu.sync_copy(dst_hbm.at[w], didx)

    for p, hp_hbm in ((0, hp_lo_hbm), (1, hp_hi_hbm)):
        def _zero_acc(k, _):
            pltpu.async_copy(
                zbuf, acc_sh.at[pl.ds(s * ROWS_PER_TILE + k * ZROWS, ZROWS)],
                gsem)
            return 0

        lax.fori_loop(0, ROWS_PER_TILE // ZROWS, _zero_acc, 0)

        def _zero_drain(k, _):
            pltpu.make_async_copy(
                zbuf, acc_sh.at[pl.ds(s * ROWS_PER_TILE + k * ZROWS, ZROWS)],
                gsem).wait()
            return 0

        lax.fori_loop(0, ROWS_PER_TILE // ZROWS, _zero_drain, 0)
        plsc.subcore_barrier()

        for q in range(NBUF - 1):
            pltpu.async_copy(hp_hbm.at[sidx.at[q]], rows.at[q], gsem)

        def _body(j, _):
            b = lax.rem(j, NBUF)
            pltpu.make_async_copy(hp_hbm.at[sidx.at[j]], rows.at[b],
                                  gsem).wait()
            pltpu.async_copy(rows.at[b], acc_sh.at[didx.at[j]], ssem,
                             add=True)

            @pl.when(j >= 1)
            def _():
                pltpu.make_async_copy(rows.at[lax.rem(j - 1, NBUF)],
                                      acc_sh.at[didx.at[j - 1]], ssem).wait()

            @pl.when(j < K - (NBUF - 1))
            def _():
                pltpu.async_copy(hp_hbm.at[sidx.at[j + NBUF - 1]],
                                 rows.at[lax.rem(j + NBUF - 1, NBUF)], gsem)

            return 0

        lax.fori_loop(0, K, _body, 0)
        pltpu.make_async_copy(rows.at[(K - 1) % NBUF],
                              acc_sh.at[didx.at[K - 1]], ssem).wait()
        plsc.subcore_barrier()

        pltpu.sync_copy(
            acc_sh.at[pl.ds(s * ROWS_PER_TILE, ROWS_PER_TILE)],
            acc_hbm.at[c, p, pl.ds(s * ROWS_PER_TILE, ROWS_PER_TILE)],
        )


@functools.cache
def _sc_kernels():
    mesh = plsc.VectorSubcoreMesh(**_mesh_kw)
    count_kernel = pl.kernel(
        _count_body,
        out_type=jax.ShapeDtypeStruct((2, NPAD, 16), jnp.float32),
        mesh=mesh,
        compiler_params=pltpu.CompilerParams(use_tc_tiling_on_sc=False),
        scratch_types=[
            pltpu.VMEM((K, CH), jnp.int32),
            pltpu.VMEM((ROWS_PER_TILE, 16), jnp.float32),
            pltpu.VMEM_SHARED((NPAD, 16), jnp.float32),
            pltpu.SemaphoreType.DMA,
        ],
    )
    agg_kernel = pl.kernel(
        _agg_body,
        out_type=jax.ShapeDtypeStruct((2, 2, NPAD, DH), jnp.float32),
        mesh=mesh,
        compiler_params=pltpu.CompilerParams(use_tc_tiling_on_sc=False),
        scratch_types=[
            pltpu.VMEM((K, CH), jnp.int32),
            pltpu.VMEM((K, CH), jnp.int32),
            pltpu.VMEM((NBUF, CH, DH), jnp.float32),
            pltpu.VMEM((ZROWS, DH), jnp.float32),
            pltpu.VMEM_SHARED((NPAD, DH), jnp.float32),
            pltpu.SemaphoreType.DMA,
            pltpu.SemaphoreType.DMA,
        ],
    )
    return count_kernel, agg_kernel


def _dense1_body(x_ref, w1_ref, cnt_ref, hp_lo_ref, hp_hi_ref, dis_ref):
    deg = cnt_ref[0][:, 0:1] + cnt_ref[1][:, 0:1] + 1.0
    dis = lax.rsqrt(deg)
    h = jnp.dot(x_ref[...], w1_ref[...], preferred_element_type=jnp.float32)
    hp = h * dis
    hp_lo_ref[...] = hp[:, :DH]
    hp_hi_ref[...] = hp[:, DH:]
    dis_ref[...] = dis


def _dense1(x_pad, W1, cnt):
    bs = 1024
    grid = NPAD // bs
    return pl.pallas_call(
        _dense1_body,
        grid=(grid,),
        in_specs=[
            pl.BlockSpec((bs, D), lambda i: (i, 0)),
            pl.BlockSpec((D, D), lambda i: (0, 0)),
            pl.BlockSpec((2, bs, 16), lambda i: (0, i, 0)),
        ],
        out_specs=[
            pl.BlockSpec((bs, DH), lambda i: (i, 0)),
            pl.BlockSpec((bs, DH), lambda i: (i, 0)),
            pl.BlockSpec((bs, 1), lambda i: (i, 0)),
        ],
        out_shape=[
            jax.ShapeDtypeStruct((NPAD, DH), jnp.float32),
            jax.ShapeDtypeStruct((NPAD, DH), jnp.float32),
            jax.ShapeDtypeStruct((NPAD, 1), jnp.float32),
        ],
    )(x_pad, W1, cnt)


def _dense2_body(acc_ref, hp_lo_ref, hp_hi_ref, dis_ref, b1_ref, gamma_ref,
                 beta_ref, w2_ref, b2_ref, out_ref):
    hp = jnp.concatenate([hp_lo_ref[...], hp_hi_ref[...]], axis=-1)
    agg = jnp.concatenate(
        [acc_ref[0, 0] + acc_ref[1, 0], acc_ref[0, 1] + acc_ref[1, 1]],
        axis=-1)
    g = (agg + hp) * dis_ref[...]
    z = jnp.maximum(g + b1_ref[...], 0.0)
    mu = jnp.mean(z, axis=-1, keepdims=True)
    zc = z - mu
    var = jnp.mean(zc * zc, axis=-1, keepdims=True)
    zn = zc * lax.rsqrt(var + 1e-5) * gamma_ref[...] + beta_ref[...]
    y = jnp.dot(zn, w2_ref[...], preferred_element_type=jnp.float32)
    out_ref[...] = jnp.maximum(y + b2_ref[...], 0.0)


def _dense2(acc, hp_lo, hp_hi, dis, b1, gamma, beta, W2, b2):
    bs = 512
    grid = NPAD // bs
    return pl.pallas_call(
        _dense2_body,
        grid=(grid,),
        in_specs=[
            pl.BlockSpec((2, 2, bs, DH), lambda i: (0, 0, i, 0)),
            pl.BlockSpec((bs, DH), lambda i: (i, 0)),
            pl.BlockSpec((bs, DH), lambda i: (i, 0)),
            pl.BlockSpec((bs, 1), lambda i: (i, 0)),
            pl.BlockSpec((D,), lambda i: (0,)),
            pl.BlockSpec((D,), lambda i: (0,)),
            pl.BlockSpec((D,), lambda i: (0,)),
            pl.BlockSpec((D, DOUT), lambda i: (0, 0)),
            pl.BlockSpec((DOUT,), lambda i: (0,)),
        ],
        out_specs=pl.BlockSpec((bs, DOUT), lambda i: (i, 0)),
        out_shape=jax.ShapeDtypeStruct((NPAD, DOUT), jnp.float32),
    )(acc, hp_lo, hp_hi, dis, b1, gamma, beta, W2, b2)


def kernel(x, edge_index, W1, b1, gamma, beta, W2, b2):
    ei = edge_index.astype(jnp.int32)
    src = ei[0]
    dst = ei[1]
    pad = EPAD - E
    src3 = jnp.concatenate(
        [src, jnp.zeros((pad,), jnp.int32)]).reshape(NW, K, CH)
    trash = N + jax.lax.rem(jnp.arange(pad, dtype=jnp.int32),
                            jnp.int32(NPAD - N))
    dst3 = jnp.concatenate([dst, trash]).reshape(NW, K, CH)
    x_pad = jnp.pad(x, ((0, NPAD - N), (0, 0)))

    count_kernel, agg_kernel = _sc_kernels()
    cnt = count_kernel(dst3)
    hp_lo, hp_hi, dis = _dense1(x_pad, W1, cnt)
    acc = agg_kernel(src3, dst3, hp_lo, hp_hi)
    out = _dense2(acc, hp_lo, hp_hi, dis, b1, gamma, beta, W2, b2)
    return (out[:N], edge_index)


# speedup vs baseline: 2.0260x; 1.3599x over previous
import functools

import jax
import jax.numpy as jnp
from jax import lax
from jax.experimental import pallas as pl
from jax.experimental.pallas import tpu as pltpu
from jax.experimental.pallas import tpu_sc as plsc

N = 10000
E = 320000
D = 128
DOUT = 256

NPAD = 10240
NW = 32
CH = 64
K = 157
EPAD = NW * K * CH
ROWS_PER_TILE = NPAD // 16
NBUF = 3
ZROWS = 32

_mesh_kw = dict(core_axis_name="c", subcore_axis_name="s",
                num_cores=2, num_subcores=16)


def _count_body(dst_hbm, cnt_hbm, idx_v, buf_v, cnt_sh, sem):
    c = lax.axis_index("c")
    s = lax.axis_index("s")
    w = c * 16 + s

    def _zero_row(i, _):
        buf_v[i, :] = jnp.zeros((16,), jnp.float32)
        return 0

    lax.fori_loop(0, ROWS_PER_TILE, _zero_row, 0)
    pltpu.sync_copy(buf_v, cnt_sh.at[pl.ds(s * ROWS_PER_TILE, ROWS_PER_TILE)])

    pltpu.sync_copy(dst_hbm.at[w], idx_v)

    def _one_row(i, _):
        buf_v[i, :] = jnp.ones((16,), jnp.float32)
        return 0

    lax.fori_loop(0, CH, _one_row, 0)

    plsc.subcore_barrier()

    def _body(j, _):
        pltpu.async_copy(buf_v.at[pl.ds(0, CH)], cnt_sh.at[idx_v.at[j]],
                         sem, add=True)
        return 0

    lax.fori_loop(0, K, _body, 0)

    def _drain(j, _):
        pltpu.make_async_copy(buf_v.at[pl.ds(0, CH)],
                              cnt_sh.at[idx_v.at[j]], sem).wait()
        return 0

    lax.fori_loop(0, K, _drain, 0)

    plsc.subcore_barrier()

    pltpu.sync_copy(
        cnt_sh.at[pl.ds(s * ROWS_PER_TILE, ROWS_PER_TILE)],
        cnt_hbm.at[c, pl.ds(s * ROWS_PER_TILE, ROWS_PER_TILE)],
    )


def _agg_body(src_hbm, dst_hbm, hp_hbm, acc_hbm,
              sidx, didx, rows, zbuf, acc_sh, gsem, ssem):
    c = lax.axis_index("c")
    s = lax.axis_index("s")
    w = c * 16 + s

    def _zero_row(i, _):
        def _zero_lane(jj, _):
            zbuf[i, pl.ds(jj * 16, 16)] = jnp.zeros((16,), jnp.float32)
            return 0
        lax.fori_loop(0, D // 16, _zero_lane, 0)
        return 0

    lax.fori_loop(0, ZROWS, _zero_row, 0)

    def _zero_acc(k, _):
        pltpu.async_copy(
            zbuf, acc_sh.at[pl.ds(s * ROWS_PER_TILE + k * ZROWS, ZROWS)],
            gsem)
        return 0

    lax.fori_loop(0, ROWS_PER_TILE // ZROWS, _zero_acc, 0)

    pltpu.sync_copy(src_hbm.at[w], sidx)
    pltpu.sync_copy(dst_hbm.at[w], didx)

    def _zero_drain(k, _):
        pltpu.make_async_copy(
            zbuf, acc_sh.at[pl.ds(s * ROWS_PER_TILE + k * ZROWS, ZROWS)],
            gsem).wait()
        return 0

    lax.fori_loop(0, ROWS_PER_TILE // ZROWS, _zero_drain, 0)
    plsc.subcore_barrier()

    for q in range(NBUF - 1):
        pltpu.async_copy(hp_hbm.at[sidx.at[q]], rows.at[q], gsem)

    def _body(j, _):
        b = lax.rem(j, NBUF)
        pltpu.make_async_copy(hp_hbm.at[sidx.at[j]], rows.at[b],
                              gsem).wait()
        pltpu.async_copy(rows.at[b], acc_sh.at[didx.at[j]], ssem,
                         add=True)

        @pl.when(j >= 1)
        def _():
            pltpu.make_async_copy(rows.at[lax.rem(j - 1, NBUF)],
                                  acc_sh.at[didx.at[j - 1]], ssem).wait()

        @pl.when(j < K - (NBUF - 1))
        def _():
            pltpu.async_copy(hp_hbm.at[sidx.at[j + NBUF - 1]],
                             rows.at[lax.rem(j + NBUF - 1, NBUF)], gsem)

        return 0

    lax.fori_loop(0, K, _body, 0)
    pltpu.make_async_copy(rows.at[(K - 1) % NBUF],
                          acc_sh.at[didx.at[K - 1]], ssem).wait()
    plsc.subcore_barrier()

    pltpu.sync_copy(
        acc_sh.at[pl.ds(s * ROWS_PER_TILE, ROWS_PER_TILE)],
        acc_hbm.at[c, pl.ds(s * ROWS_PER_TILE, ROWS_PER_TILE)],
    )


@functools.cache
def _sc_kernels():
    mesh = plsc.VectorSubcoreMesh(**_mesh_kw)
    count_kernel = pl.kernel(
        _count_body,
        out_type=jax.ShapeDtypeStruct((2, NPAD, 16), jnp.float32),
        mesh=mesh,
        compiler_params=pltpu.CompilerParams(use_tc_tiling_on_sc=False),
        scratch_types=[
            pltpu.VMEM((K, CH), jnp.int32),
            pltpu.VMEM((ROWS_PER_TILE, 16), jnp.float32),
            pltpu.VMEM_SHARED((NPAD, 16), jnp.float32),
            pltpu.SemaphoreType.DMA,
        ],
    )
    agg_kernel = pl.kernel(
        _agg_body,
        out_type=jax.ShapeDtypeStruct((2, NPAD, D), jnp.float32),
        mesh=mesh,
        compiler_params=pltpu.CompilerParams(use_tc_tiling_on_sc=False),
        scratch_types=[
            pltpu.VMEM((K, CH), jnp.int32),
            pltpu.VMEM((K, CH), jnp.int32),
            pltpu.VMEM((NBUF, CH, D), jnp.float32),
            pltpu.VMEM((ZROWS, D), jnp.float32),
            pltpu.VMEM_SHARED((NPAD, D), jnp.float32),
            pltpu.SemaphoreType.DMA,
            pltpu.SemaphoreType.DMA,
        ],
    )
    return count_kernel, agg_kernel


def _dense1_body(x_ref, w1_ref, cnt_ref, hp_ref, dis_ref):
    deg = cnt_ref[0][:, 0:1] + cnt_ref[1][:, 0:1] + 1.0
    dis = lax.rsqrt(deg)
    h = jnp.dot(x_ref[...], w1_ref[...], preferred_element_type=jnp.float32)
    hp_ref[...] = h * dis
    dis_ref[...] = dis


def _dense1(x_pad, W1, cnt):
    bs = 1024
    grid = NPAD // bs
    return pl.pallas_call(
        _dense1_body,
        grid=(grid,),
        in_specs=[
            pl.BlockSpec((bs, D), lambda i: (i, 0)),
            pl.BlockSpec((D, D), lambda i: (0, 0)),
            pl.BlockSpec((2, bs, 16), lambda i: (0, i, 0)),
        ],
        out_specs=[
            pl.BlockSpec((bs, D), lambda i: (i, 0)),
            pl.BlockSpec((bs, 1), lambda i: (i, 0)),
        ],
        out_shape=[
            jax.ShapeDtypeStruct((NPAD, D), jnp.float32),
            jax.ShapeDtypeStruct((NPAD, 1), jnp.float32),
        ],
    )(x_pad, W1, cnt)


def _dense2_body(acc_ref, hp_ref, dis_ref, b1_ref, gamma_ref,
                 beta_ref, w2_ref, b2_ref, out_ref):
    agg = acc_ref[0] + acc_ref[1]
    g = (agg + hp_ref[...]) * dis_ref[...]
    z = jnp.maximum(g + b1_ref[...], 0.0)
    mu = jnp.mean(z, axis=-1, keepdims=True)
    zc = z - mu
    var = jnp.mean(zc * zc, axis=-1, keepdims=True)
    zn = zc * lax.rsqrt(var + 1e-5) * gamma_ref[...] + beta_ref[...]
    y = jnp.dot(zn, w2_ref[...], preferred_element_type=jnp.float32)
    out_ref[...] = jnp.maximum(y + b2_ref[...], 0.0)


def _dense2(acc, hp, dis, b1, gamma, beta, W2, b2):
    bs = 512
    grid = NPAD // bs
    return pl.pallas_call(
        _dense2_body,
        grid=(grid,),
        in_specs=[
            pl.BlockSpec((2, bs, D), lambda i: (0, i, 0)),
            pl.BlockSpec((bs, D), lambda i: (i, 0)),
            pl.BlockSpec((bs, 1), lambda i: (i, 0)),
            pl.BlockSpec((D,), lambda i: (0,)),
            pl.BlockSpec((D,), lambda i: (0,)),
            pl.BlockSpec((D,), lambda i: (0,)),
            pl.BlockSpec((D, DOUT), lambda i: (0, 0)),
            pl.BlockSpec((DOUT,), lambda i: (0,)),
        ],
        out_specs=pl.BlockSpec((bs, DOUT), lambda i: (i, 0)),
        out_shape=jax.ShapeDtypeStruct((NPAD, DOUT), jnp.float32),
    )(acc, hp, dis, b1, gamma, beta, W2, b2)


def kernel(x, edge_index, W1, b1, gamma, beta, W2, b2):
    ei = edge_index.astype(jnp.int32)
    src = ei[0]
    dst = ei[1]
    pad = EPAD - E
    src3 = jnp.concatenate(
        [src, jnp.zeros((pad,), jnp.int32)]).reshape(NW, K, CH)
    trash = N + jax.lax.rem(jnp.arange(pad, dtype=jnp.int32),
                            jnp.int32(NPAD - N))
    dst3 = jnp.concatenate([dst, trash]).reshape(NW, K, CH)
    x_pad = jnp.pad(x, ((0, NPAD - N), (0, 0)))

    count_kernel, agg_kernel = _sc_kernels()
    cnt = count_kernel(dst3)
    hp, dis = _dense1(x_pad, W1, cnt)
    acc = agg_kernel(src3, dst3, hp)
    out = _dense2(acc, hp, dis, b1, gamma, beta, W2, b2)
    return (out[:N], edge_index)
